# Initial kernel scaffold; baseline (speedup 1.0000x reference)
import jax
import jax.numpy as jnp
from jax.experimental import pallas as pl


def kernel(x, edge_index, W1_l, b1_l, W1_r, W2_l, b2_l, W2_r):
    raise NotImplementedError("write your pallas kernel here")


# speedup vs baseline: 3.9465x; 3.9465x over previous
import functools

import jax
import jax.numpy as jnp
from jax import lax
from jax.experimental import pallas as pl
from jax.experimental.pallas import tpu as pltpu
from jax.experimental.pallas import tpu_sc as plsc

N = 10000
D = 128
E = 320000
NW = 32
CHUNK = 128
C = 79
EPW = C * CHUNK
EPAD = NW * EPW
NSLICE = 632
NPAD = 16 * NSLICE

_MESH = dict(core_axis_name="c", subcore_axis_name="s")
_SLICES = [(0, 128), (128, 128), (256, 128), (384, 128), (512, 120)]


@functools.partial(
    pl.kernel,
    mesh=plsc.VectorSubcoreMesh(**_MESH),
    out_type=jax.ShapeDtypeStruct((2 * NPAD, D), jnp.float32),
    scratch_types=[
        pltpu.VMEM((1, CHUNK), jnp.int32),
        pltpu.VMEM((1, CHUNK), jnp.int32),
        pltpu.VMEM((CHUNK, D), jnp.float32),
        pltpu.VMEM_SHARED((NPAD, D), jnp.float32),
        pltpu.SemaphoreType.DMA,
    ],
)
def _seg_sum(table, src_f, dst_f, zacc, acc_out,
             src_v, dst_v, rows_v, acc_sh, sem):
    c = lax.axis_index("c")
    s = lax.axis_index("s")
    wid = c * 16 + s
    r0 = s * NSLICE
    base = wid * EPW
    for t, sz in _SLICES:
        pltpu.sync_copy(zacc.at[pl.ds(r0 + t, sz)], rows_v.at[pl.ds(0, sz)])
        pltpu.sync_copy(rows_v.at[pl.ds(0, sz)], acc_sh.at[pl.ds(r0 + t, sz)])
    plsc.subcore_barrier()

    def body(i, carry):
        off = base + i * CHUNK
        pltpu.sync_copy(src_f.at[pl.ds(off, CHUNK)], src_v.at[0])
        pltpu.sync_copy(dst_f.at[pl.ds(off, CHUNK)], dst_v.at[0])
        pltpu.async_copy(table.at[src_v.at[0]], rows_v, sem).wait()
        pltpu.sync_copy(rows_v, acc_sh.at[dst_v.at[0]], add=True)
        return carry

    lax.fori_loop(0, C, body, 0)

    plsc.subcore_barrier()
    o0 = c * NPAD + s * NSLICE
    for t, sz in _SLICES:
        pltpu.sync_copy(acc_sh.at[pl.ds(r0 + t, sz)], rows_v.at[pl.ds(0, sz)])
        pltpu.sync_copy(rows_v.at[pl.ds(0, sz)], acc_out.at[pl.ds(o0 + t, sz)])


@functools.partial(
    pl.kernel,
    mesh=plsc.VectorSubcoreMesh(**_MESH),
    out_type=jax.ShapeDtypeStruct((2 * NPAD, D), jnp.float32),
    scratch_types=[
        pltpu.VMEM((1, CHUNK), jnp.int32),
        pltpu.VMEM((CHUNK, D), jnp.float32),
        pltpu.VMEM_SHARED((NPAD, D), jnp.float32),
    ],
)
def _deg_sum(dst_f, zacc, ones, deg_out, dst_v, ones_v, deg_sh):
    c = lax.axis_index("c")
    s = lax.axis_index("s")
    wid = c * 16 + s
    r0 = s * NSLICE
    base = wid * EPW
    for t, sz in _SLICES:
        pltpu.sync_copy(zacc.at[pl.ds(r0 + t, sz)], ones_v.at[pl.ds(0, sz)])
        pltpu.sync_copy(ones_v.at[pl.ds(0, sz)], deg_sh.at[pl.ds(r0 + t, sz)])
    pltpu.sync_copy(ones, ones_v)
    plsc.subcore_barrier()

    def body(i, carry):
        off = base + i * CHUNK
        pltpu.sync_copy(dst_f.at[pl.ds(off, CHUNK)], dst_v.at[0])
        pltpu.sync_copy(ones_v, deg_sh.at[dst_v.at[0]], add=True)
        return carry

    lax.fori_loop(0, C, body, 0)

    plsc.subcore_barrier()
    o0 = c * NPAD + s * NSLICE
    for t, sz in _SLICES:
        pltpu.sync_copy(deg_sh.at[pl.ds(r0 + t, sz)], ones_v.at[pl.ds(0, sz)])
        pltpu.sync_copy(ones_v.at[pl.ds(0, sz)], deg_out.at[pl.ds(o0 + t, sz)])


_MMB = 2000


def _mm2_body(x_ref, wl_ref, wr_ref, p_ref, r_ref):
    x = x_ref[...]
    p_ref[...] = jnp.dot(x, wl_ref[...], preferred_element_type=jnp.float32)
    r_ref[...] = jnp.dot(x, wr_ref[...], preferred_element_type=jnp.float32)


def _mm2(x, wl, wr):
    return pl.pallas_call(
        _mm2_body,
        grid=(N // _MMB,),
        in_specs=[
            pl.BlockSpec((_MMB, D), lambda i: (i, 0)),
            pl.BlockSpec((D, D), lambda i: (0, 0)),
            pl.BlockSpec((D, D), lambda i: (0, 0)),
        ],
        out_specs=[pl.BlockSpec((_MMB, D), lambda i: (i, 0))] * 2,
        out_shape=[jax.ShapeDtypeStruct((N, D), jnp.float32)] * 2,
    )(x, wl, wr)


def _fuse_body(acc_ref, deg_ref, r1_ref, b_ref, wl_ref, wr_ref, p2_ref, r2_ref):
    a = acc_ref[0] + acc_ref[1]
    dcol = deg_ref[0, :, :1] + deg_ref[1, :, :1]
    inv = 1.0 / jnp.maximum(dcol, 1.0)
    h = jnp.maximum(a * inv + b_ref[...] + r1_ref[...], 0.0)
    p2_ref[...] = jnp.dot(h, wl_ref[...], preferred_element_type=jnp.float32)
    r2_ref[...] = jnp.dot(h, wr_ref[...], preferred_element_type=jnp.float32)


def _fuse(acc, deg, r1, b1, wl, wr):
    return pl.pallas_call(
        _fuse_body,
        grid=(N // _MMB,),
        in_specs=[
            pl.BlockSpec((2, _MMB, D), lambda i: (0, i, 0)),
            pl.BlockSpec((2, _MMB, D), lambda i: (0, i, 0)),
            pl.BlockSpec((_MMB, D), lambda i: (i, 0)),
            pl.BlockSpec((1, D), lambda i: (0, 0)),
            pl.BlockSpec((D, D), lambda i: (0, 0)),
            pl.BlockSpec((D, D), lambda i: (0, 0)),
        ],
        out_specs=[pl.BlockSpec((_MMB, D), lambda i: (i, 0))] * 2,
        out_shape=[jax.ShapeDtypeStruct((N, D), jnp.float32)] * 2,
    )(acc, deg, r1, b1, wl, wr)


def _final_body(acc_ref, deg_ref, r2_ref, b_ref, out_ref):
    a = acc_ref[0] + acc_ref[1]
    dcol = deg_ref[0, :, :1] + deg_ref[1, :, :1]
    inv = 1.0 / jnp.maximum(dcol, 1.0)
    out_ref[...] = a * inv + b_ref[...] + r2_ref[...]


def _final(acc, deg, r2, b2):
    return pl.pallas_call(
        _final_body,
        grid=(N // _MMB,),
        in_specs=[
            pl.BlockSpec((2, _MMB, D), lambda i: (0, i, 0)),
            pl.BlockSpec((2, _MMB, D), lambda i: (0, i, 0)),
            pl.BlockSpec((_MMB, D), lambda i: (i, 0)),
            pl.BlockSpec((1, D), lambda i: (0, 0)),
        ],
        out_specs=pl.BlockSpec((_MMB, D), lambda i: (i, 0)),
        out_shape=jax.ShapeDtypeStruct((N, D), jnp.float32),
    )(acc, deg, r2, b2)


def kernel(x, edge_index, W1_l, b1_l, W1_r, W2_l, b2_l, W2_r):
    pad = EPAD - E
    src = jnp.concatenate(
        [edge_index[0].astype(jnp.int32), jnp.zeros((pad,), jnp.int32)])
    dst = jnp.concatenate(
        [edge_index[1].astype(jnp.int32), jnp.full((pad,), N, jnp.int32)])
    zacc = jnp.zeros((NPAD, D), jnp.float32)
    ones = jnp.ones((CHUNK, D), jnp.float32)
    b1 = b1_l.reshape(1, D)
    b2 = b2_l.reshape(1, D)

    p1, r1 = _mm2(x, W1_l, W1_r)
    acc1 = _seg_sum(p1, src, dst, zacc).reshape(2, NPAD, D)
    degp = _deg_sum(dst, zacc, ones).reshape(2, NPAD, D)
    p2, r2 = _fuse(acc1, degp, r1, b1, W2_l, W2_r)
    acc2 = _seg_sum(p2, src, dst, zacc).reshape(2, NPAD, D)
    return _final(acc2, degp, r2, b2)
